# Initial kernel scaffold; baseline (speedup 1.0000x reference)
import jax
import jax.numpy as jnp
from jax.experimental import pallas as pl


def kernel(input, edge_index, cell_dropout, layer_dropout, node_lastlayer, stage1_flag, W, b):
    raise NotImplementedError("write your pallas kernel here")


# speedup vs baseline: 4.0503x; 4.0503x over previous
import functools

import jax
import jax.numpy as jnp
from jax import lax
from jax.experimental import pallas as pl
from jax.experimental.pallas import tpu as pltpu
from jax.experimental.pallas import tpu_sc as plsc

N = 10000
E = 320000
D = 128

NC = 2
NS = 16
NW = NC * NS
CHUNK = 128
NCHUNK = -(-E // (NW * CHUNK))
EPW = NCHUNK * CHUNK
EPAD = EPW * NW
NACC = 10112
RPT = NACC // NS
LAST = N - 15 * RPT


def _sc_segment_sum(x, src, dst, zero_init):
    mesh = plsc.VectorSubcoreMesh(core_axis_name="c", subcore_axis_name="s")

    @functools.partial(
        pl.kernel,
        mesh=mesh,
        out_type=jax.ShapeDtypeStruct((2 * N, D), jnp.float32),
        scratch_types=[
            pltpu.VMEM((CHUNK,), jnp.int32),
            pltpu.VMEM((CHUNK,), jnp.int32),
            pltpu.VMEM((CHUNK, D), jnp.float32),
            pltpu.VMEM_SHARED((NACC, D), jnp.float32),
            pltpu.SemaphoreType.DMA,
        ],
    )
    def k(x_hbm, src_hbm, dst_hbm, zero_hbm, out_hbm,
          src_v, dst_v, rows_v, acc, sem):
        c = lax.axis_index("c")
        s = lax.axis_index("s")
        w = s * NC + c

        pltpu.sync_copy(zero_hbm, acc.at[pl.ds(s * RPT, RPT)])
        plsc.subcore_barrier()

        ebase = w * EPW

        def body(kk, carry):
            base = pl.multiple_of(ebase + kk * CHUNK, 8)
            pltpu.sync_copy(src_hbm.at[pl.ds(base, CHUNK)], src_v)
            pltpu.sync_copy(dst_hbm.at[pl.ds(base, CHUNK)], dst_v)
            pltpu.async_copy(x_hbm.at[src_v], rows_v, sem).wait()
            pltpu.sync_copy(rows_v, acc.at[dst_v], add=True)
            return carry

        lax.fori_loop(0, NCHUNK, body, 0)
        plsc.subcore_barrier()

        @pl.when(s < NS - 1)
        def _():
            pltpu.sync_copy(acc.at[pl.ds(s * RPT, RPT)],
                            out_hbm.at[pl.ds(c * N + s * RPT, RPT)])

        @pl.when(s == NS - 1)
        def _():
            pltpu.sync_copy(acc.at[pl.ds((NS - 1) * RPT, LAST)],
                            out_hbm.at[pl.ds(c * N + (NS - 1) * RPT, LAST)])

    return k(x, src, dst, zero_init)


BM = 1000


def _tc_tail(x, partials, W, b2):
    def body(x_ref, p0_ref, p1_ref, w_ref, b_ref, o_ref):
        a = p0_ref[...] + p1_ref[...]
        h = jnp.dot(a, w_ref[...], preferred_element_type=jnp.float32)
        o_ref[...] = x_ref[...] + jnp.maximum(h + b_ref[...], 0.0)

    return pl.pallas_call(
        body,
        grid=(N // BM,),
        in_specs=[
            pl.BlockSpec((BM, D), lambda i: (i, 0)),
            pl.BlockSpec((BM, D), lambda i: (i, 0)),
            pl.BlockSpec((BM, D), lambda i: (i + N // BM, 0)),
            pl.BlockSpec((D, D), lambda i: (0, 0)),
            pl.BlockSpec((1, D), lambda i: (0, 0)),
        ],
        out_specs=pl.BlockSpec((BM, D), lambda i: (i, 0)),
        out_shape=jax.ShapeDtypeStruct((N, D), jnp.float32),
    )(x, partials, partials, W, b2)


def kernel(input, edge_index, cell_dropout, layer_dropout, node_lastlayer,
           stage1_flag, W, b):
    pad = EPAD - E
    src = jnp.concatenate(
        [edge_index[0], jnp.zeros((pad,), dtype=jnp.int32)])
    dst = jnp.concatenate(
        [edge_index[1], jnp.full((pad,), N, dtype=jnp.int32)])
    zero_init = jnp.zeros((RPT, D), dtype=jnp.float32)

    partials = _sc_segment_sum(input, src, dst, zero_init)
    return _tc_tail(input, partials, W, b.reshape(1, D))
